# baseline (speedup 1.0000x reference)
import functools
import math

import jax
import jax.numpy as jnp
from jax import lax
from jax.experimental import pallas as pl
from jax.experimental.pallas import tpu as pltpu
from jax.experimental.pallas import tpu_sc as plsc

B = 16384
NF = 26
HF = NF // 2
V = 100000
D = 48
H = 128
DP = 128

VP = 100096
BLKV = 2176
NBV = VP // BLKV

NC = 2
NS = 16
NW = NC * NS
ROWS_PER_W = B // NW
SUPER = 16
NSUP = ROWS_PER_W // SUPER

GA = 10
GB = HF - GA


def _pack_table(tables_t, i0, ni):
    def body(x1_ref, x2_ref, o_ref):
        o_ref[:, :D] = jnp.transpose(x1_ref[0], (1, 0))
        o_ref[:, D:2 * D] = jnp.transpose(x2_ref[0], (1, 0))

    return pl.pallas_call(
        body,
        grid=(ni, NBV),
        in_specs=[
            pl.BlockSpec((1, D, BLKV), lambda i, v: (i + i0, 0, v)),
            pl.BlockSpec((1, D, BLKV), lambda i, v: (i + i0 + HF, 0, v)),
        ],
        out_specs=pl.BlockSpec((BLKV, DP), lambda i, v: (i * NBV + v, 0)),
        out_shape=jax.ShapeDtypeStruct((ni * VP, DP), jnp.float32),
    )(tables_t, tables_t)


def _sc_gather_sum(flat_idx, tabp, npair):
    nfg = 2 * npair
    idxc = SUPER * nfg
    nstr = 2 if idxc <= 256 else 4
    slen = idxc // nstr
    assert slen <= 128 and slen % 8 == 0
    mesh = plsc.VectorSubcoreMesh(core_axis_name="c", subcore_axis_name="s")

    @functools.partial(
        pl.kernel,
        mesh=mesh,
        out_type=jax.ShapeDtypeStruct((B, DP), jnp.float32),
        scratch_types=[
            pltpu.VMEM((idxc,), jnp.int32),
            pltpu.VMEM((idxc,), jnp.int32),
            pltpu.VMEM((idxc, DP), jnp.float32),
            pltpu.VMEM((idxc, DP), jnp.float32),
            pltpu.VMEM((SUPER, DP), jnp.float32),
            pltpu.VMEM((SUPER, DP), jnp.float32),
            pltpu.SemaphoreType.DMA,
            pltpu.SemaphoreType.DMA,
            pltpu.SemaphoreType.DMA,
            pltpu.SemaphoreType.DMA,
            pltpu.SemaphoreType.DMA,
            pltpu.SemaphoreType.DMA,
        ],
    )
    def k(idx_hbm, tab_hbm, out_hbm,
          i0, i1, g0, g1, o0, o1, si0, si1, sg0, sg1, so0, so1):
        wid = lax.axis_index("s") * NC + lax.axis_index("c")
        wbase = wid * (NSUP * idxc)
        rbase = wid * ROWS_PER_W
        ibufs, gbufs, obufs = (i0, i1), (g0, g1), (o0, o1)
        sis, sgs, sos = (si0, si1), (sg0, sg1), (so0, so1)

        def idx_start(s, buf, sem):
            pltpu.async_copy(idx_hbm.at[pl.ds(wbase + s * idxc, idxc)],
                             buf, sem)

        def idx_wait(s, buf, sem):
            pltpu.make_async_copy(
                idx_hbm.at[pl.ds(wbase + s * idxc, idxc)], buf, sem).wait()

        def gathers_start(ibuf, gbuf, sem):
            for t in range(nstr):
                pltpu.async_copy(
                    tab_hbm.at[ibuf.at[pl.ds(t * slen, slen)]],
                    gbuf.at[pl.ds(t * slen, slen)], sem)

        def gathers_wait(ibuf, gbuf, sem):
            for t in range(nstr):
                pltpu.make_async_copy(
                    tab_hbm.at[ibuf.at[pl.ds(t * slen, slen)]],
                    gbuf.at[pl.ds(t * slen, slen)], sem).wait()

        def out_start(s, obuf, sem):
            pltpu.async_copy(obuf,
                             out_hbm.at[pl.ds(rbase + s * SUPER, SUPER)], sem)

        def out_wait(s, obuf, sem):
            pltpu.make_async_copy(
                obuf, out_hbm.at[pl.ds(rbase + s * SUPER, SUPER)], sem).wait()

        def reduce_step(gbuf, obuf):
            def red_row(r, c):
                qb = r * nfg
                accs = [jnp.zeros((16,), jnp.float32)
                        for _ in range(D // 16)]
                for f in range(nfg):
                    q = qb + f
                    off = 0 if f < npair else D
                    for j in range(D // 16):
                        accs[j] = accs[j] + gbuf[q, pl.ds(off + j * 16, 16)]
                for j in range(D // 16):
                    obuf[r, pl.ds(j * 16, 16)] = accs[j]
                return c
            lax.fori_loop(0, SUPER, red_row, 0)

        pltpu.sync_copy(idx_hbm.at[pl.ds(wbase, idxc)], i0)
        gathers_start(i0, g0, sg0)
        idx_start(1, i1, si1)

        def body2(h, c):
            for par in range(2):
                s = h * 2 + par
                ib, gb, ob = ibufs[par], gbufs[par], obufs[par]
                gathers_wait(ib, gb, sgs[par])

                @pl.when(s + 1 < NSUP)
                def _():
                    idx_wait(s + 1, ibufs[1 - par], sis[1 - par])
                    gathers_start(ibufs[1 - par], gbufs[1 - par],
                                  sgs[1 - par])

                @pl.when(s + 2 < NSUP)
                def _():
                    idx_start(s + 2, ib, sis[par])

                @pl.when(s >= 2)
                def _():
                    out_wait(s - 2, ob, sos[par])

                reduce_step(gb, ob)
                out_start(s, ob, sos[par])
            return c

        lax.fori_loop(0, NSUP // 2, body2, 0)
        out_wait(NSUP - 2, o0, so0)
        out_wait(NSUP - 1, o1, so1)

    return k(flat_idx, tabp)


def _tc_proj(sa, sb, W, b2):
    TILE = 2048

    def body(sa_ref, sb_ref, w_ref, b_ref, o_ref):
        s = sa_ref[:, :D] + sb_ref[:, :D]
        x = jnp.dot(s, w_ref[...],
                    preferred_element_type=jnp.float32) + b_ref[...]
        o_ref[...] = 0.5 * x * (1.0 + lax.erf(x * (1.0 / math.sqrt(2.0))))

    return pl.pallas_call(
        body,
        grid=(B // TILE,),
        in_specs=[
            pl.BlockSpec((TILE, DP), lambda i: (i, 0)),
            pl.BlockSpec((TILE, DP), lambda i: (i, 0)),
            pl.BlockSpec((D, H), lambda i: (0, 0)),
            pl.BlockSpec((1, H), lambda i: (0, 0)),
        ],
        out_specs=pl.BlockSpec((TILE, H), lambda i: (i, 0)),
        out_shape=jax.ShapeDtypeStruct((B, H), jnp.float32),
    )(sa, sb, W, b2)


def _group_idx(e, lo, ni):
    ew = jnp.concatenate([e[:, lo:lo + ni], e[:, lo + HF:lo + HF + ni]],
                         axis=1)
    foffs = jnp.concatenate([jnp.arange(ni, dtype=jnp.int32)] * 2) * VP
    return (ew + foffs[None, :]).reshape(B * 2 * ni)


def kernel(e, tables, W, b):
    tables_t = jnp.transpose(tables, (0, 2, 1))
    idx_a = _group_idx(e, 0, GA)
    idx_b = _group_idx(e, GA, GB)
    tab_a = _pack_table(tables_t, 0, GA)
    sa = _sc_gather_sum(idx_a, tab_a, GA)
    tab_b = _pack_table(tables_t, GA, GB)
    sb = _sc_gather_sum(idx_b, tab_b, GB)
    return _tc_proj(sa, sb, W, b.reshape(1, H))


# speedup vs baseline: 1.2224x; 1.2224x over previous
import functools
import math

import jax
import jax.numpy as jnp
from jax import lax
from jax.experimental import pallas as pl
from jax.experimental.pallas import tpu as pltpu
from jax.experimental.pallas import tpu_sc as plsc

B = 16384
NF = 26
HF = NF // 2
V = 100000
D = 48
H = 128
DP = 128

VP = 100096
BLKV = 4352
NBV = VP // BLKV

NC = 2
NS = 16
NW = NC * NS
ROWS_PER_W = B // NW
SUPER = 16
NSUP = ROWS_PER_W // SUPER

GA = 10
GB = HF - GA


def _pack_table(tables_t, i0, ni):
    def body(x1_ref, x2_ref, o_ref):
        o_ref[:, :D] = jnp.transpose(x1_ref[0], (1, 0))
        o_ref[:, D:2 * D] = jnp.transpose(x2_ref[0], (1, 0))

    return pl.pallas_call(
        body,
        grid=(ni, NBV),
        in_specs=[
            pl.BlockSpec((1, D, BLKV), lambda i, v: (i + i0, 0, v)),
            pl.BlockSpec((1, D, BLKV), lambda i, v: (i + i0 + HF, 0, v)),
        ],
        out_specs=pl.BlockSpec((BLKV, DP), lambda i, v: (i * NBV + v, 0)),
        out_shape=jax.ShapeDtypeStruct((ni * VP, DP), jnp.float32),
    )(tables_t, tables_t)


def _sc_gather_sum(flat_idx, tabp, npair):
    nfg = 2 * npair
    idxc = SUPER * nfg
    nstr = 2 if idxc <= 256 else 4
    slen = idxc // nstr
    assert slen <= 128 and slen % 8 == 0
    mesh = plsc.VectorSubcoreMesh(core_axis_name="c", subcore_axis_name="s")

    @functools.partial(
        pl.kernel,
        mesh=mesh,
        out_type=jax.ShapeDtypeStruct((B, DP), jnp.float32),
        scratch_types=[
            pltpu.VMEM((idxc,), jnp.int32),
            pltpu.VMEM((idxc,), jnp.int32),
            pltpu.VMEM((idxc, DP), jnp.float32),
            pltpu.VMEM((idxc, DP), jnp.float32),
            pltpu.VMEM((SUPER, DP), jnp.float32),
            pltpu.VMEM((SUPER, DP), jnp.float32),
            pltpu.SemaphoreType.DMA,
            pltpu.SemaphoreType.DMA,
            pltpu.SemaphoreType.DMA,
            pltpu.SemaphoreType.DMA,
            pltpu.SemaphoreType.DMA,
            pltpu.SemaphoreType.DMA,
        ],
    )
    def k(idx_hbm, tab_hbm, out_hbm,
          i0, i1, g0, g1, o0, o1, si0, si1, sg0, sg1, so0, so1):
        wid = lax.axis_index("s") * NC + lax.axis_index("c")
        wbase = wid * (NSUP * idxc)
        rbase = wid * ROWS_PER_W
        ibufs, gbufs, obufs = (i0, i1), (g0, g1), (o0, o1)
        sis, sgs, sos = (si0, si1), (sg0, sg1), (so0, so1)

        def idx_start(s, buf, sem):
            pltpu.async_copy(idx_hbm.at[pl.ds(wbase + s * idxc, idxc)],
                             buf, sem)

        def idx_wait(s, buf, sem):
            pltpu.make_async_copy(
                idx_hbm.at[pl.ds(wbase + s * idxc, idxc)], buf, sem).wait()

        def gathers_start(ibuf, gbuf, sem):
            for t in range(nstr):
                pltpu.async_copy(
                    tab_hbm.at[ibuf.at[pl.ds(t * slen, slen)]],
                    gbuf.at[pl.ds(t * slen, slen)], sem)

        def gathers_wait(ibuf, gbuf, sem):
            for t in range(nstr):
                pltpu.make_async_copy(
                    tab_hbm.at[ibuf.at[pl.ds(t * slen, slen)]],
                    gbuf.at[pl.ds(t * slen, slen)], sem).wait()

        def out_start(s, obuf, sem):
            pltpu.async_copy(obuf,
                             out_hbm.at[pl.ds(rbase + s * SUPER, SUPER)], sem)

        def out_wait(s, obuf, sem):
            pltpu.make_async_copy(
                obuf, out_hbm.at[pl.ds(rbase + s * SUPER, SUPER)], sem).wait()

        def reduce_step(gbuf, obuf):
            def red_row(r, c):
                qb = r * nfg
                accs = [jnp.zeros((16,), jnp.float32)
                        for _ in range(D // 16)]
                for f in range(nfg):
                    q = qb + f
                    off = 0 if f < npair else D
                    for j in range(D // 16):
                        accs[j] = accs[j] + gbuf[q, pl.ds(off + j * 16, 16)]
                for j in range(D // 16):
                    obuf[r, pl.ds(j * 16, 16)] = accs[j]
                return c
            lax.fori_loop(0, SUPER, red_row, 0)

        pltpu.sync_copy(idx_hbm.at[pl.ds(wbase, idxc)], i0)
        gathers_start(i0, g0, sg0)
        idx_start(1, i1, si1)

        def body2(h, c):
            for par in range(2):
                s = h * 2 + par
                ib, gb, ob = ibufs[par], gbufs[par], obufs[par]
                gathers_wait(ib, gb, sgs[par])

                @pl.when(s + 1 < NSUP)
                def _():
                    idx_wait(s + 1, ibufs[1 - par], sis[1 - par])
                    gathers_start(ibufs[1 - par], gbufs[1 - par],
                                  sgs[1 - par])

                @pl.when(s + 2 < NSUP)
                def _():
                    idx_start(s + 2, ib, sis[par])

                @pl.when(s >= 2)
                def _():
                    out_wait(s - 2, ob, sos[par])

                reduce_step(gb, ob)
                out_start(s, ob, sos[par])
            return c

        lax.fori_loop(0, NSUP // 2, body2, 0)
        out_wait(NSUP - 2, o0, so0)
        out_wait(NSUP - 1, o1, so1)

    return k(flat_idx, tabp)


def _tc_proj(sa, sb, W, b2):
    TILE = 2048

    def body(sa_ref, sb_ref, w_ref, b_ref, o_ref):
        s = sa_ref[:, :D] + sb_ref[:, :D]
        x = jnp.dot(s, w_ref[...],
                    preferred_element_type=jnp.float32) + b_ref[...]
        o_ref[...] = 0.5 * x * (1.0 + lax.erf(x * (1.0 / math.sqrt(2.0))))

    return pl.pallas_call(
        body,
        grid=(B // TILE,),
        in_specs=[
            pl.BlockSpec((TILE, DP), lambda i: (i, 0)),
            pl.BlockSpec((TILE, DP), lambda i: (i, 0)),
            pl.BlockSpec((D, H), lambda i: (0, 0)),
            pl.BlockSpec((1, H), lambda i: (0, 0)),
        ],
        out_specs=pl.BlockSpec((TILE, H), lambda i: (i, 0)),
        out_shape=jax.ShapeDtypeStruct((B, H), jnp.float32),
    )(sa, sb, W, b2)


def _group_idx(e, lo, ni):
    ew = jnp.concatenate([e[:, lo:lo + ni], e[:, lo + HF:lo + HF + ni]],
                         axis=1)
    foffs = jnp.concatenate([jnp.arange(ni, dtype=jnp.int32)] * 2) * VP
    return (ew + foffs[None, :]).reshape(B * 2 * ni)


def kernel(e, tables, W, b):
    tables_t = jnp.transpose(tables, (0, 2, 1))
    idx_a = _group_idx(e, 0, GA)
    idx_b = _group_idx(e, GA, GB)
    tab_a = _pack_table(tables_t, 0, GA)
    sa = _sc_gather_sum(idx_a, tab_a, GA)
    tab_b = _pack_table(tables_t, GA, GB)
    sb = _sc_gather_sum(idx_b, tab_b, GB)
    return _tc_proj(sa, sb, W, b.reshape(1, H))


# speedup vs baseline: 1.2919x; 1.0569x over previous
import functools
import math

import jax
import jax.numpy as jnp
from jax import lax
from jax.experimental import pallas as pl
from jax.experimental.pallas import tpu as pltpu
from jax.experimental.pallas import tpu_sc as plsc

B = 16384
NF = 26
HF = NF // 2
V = 100000
D = 48
H = 128
DP = 128

VP = 100096
BLKV = 5888
NBV = VP // BLKV

NC = 2
NS = 16
NW = NC * NS
ROWS_PER_W = B // NW
SUPER = 16
NSUP = ROWS_PER_W // SUPER

GA = 10
GB = HF - GA


def _pack_table(tables_t, i0, ni):
    def body(x1_ref, x2_ref, o_ref):
        o_ref[:, :D] = jnp.transpose(x1_ref[0], (1, 0))
        o_ref[:, D:2 * D] = jnp.transpose(x2_ref[0], (1, 0))

    return pl.pallas_call(
        body,
        grid=(ni, NBV),
        in_specs=[
            pl.BlockSpec((1, D, BLKV), lambda i, v: (i + i0, 0, v)),
            pl.BlockSpec((1, D, BLKV), lambda i, v: (i + i0 + HF, 0, v)),
        ],
        out_specs=pl.BlockSpec((BLKV, DP), lambda i, v: (i * NBV + v, 0)),
        out_shape=jax.ShapeDtypeStruct((ni * VP, DP), jnp.float32),
    )(tables_t, tables_t)


def _sc_gather_sum(flat_idx, tabp, npair):
    nfg = 2 * npair
    idxc = SUPER * nfg
    nstr = 2 if idxc <= 256 else 4
    slen = idxc // nstr
    assert slen <= 128 and slen % 8 == 0
    mesh = plsc.VectorSubcoreMesh(core_axis_name="c", subcore_axis_name="s")

    @functools.partial(
        pl.kernel,
        mesh=mesh,
        out_type=jax.ShapeDtypeStruct((B, DP), jnp.float32),
        scratch_types=[
            pltpu.VMEM((idxc,), jnp.int32),
            pltpu.VMEM((idxc,), jnp.int32),
            pltpu.VMEM((idxc, DP), jnp.float32),
            pltpu.VMEM((idxc, DP), jnp.float32),
            pltpu.VMEM((SUPER, DP), jnp.float32),
            pltpu.VMEM((SUPER, DP), jnp.float32),
            pltpu.SemaphoreType.DMA,
            pltpu.SemaphoreType.DMA,
            pltpu.SemaphoreType.DMA,
            pltpu.SemaphoreType.DMA,
            pltpu.SemaphoreType.DMA,
            pltpu.SemaphoreType.DMA,
        ],
    )
    def k(idx_hbm, tab_hbm, out_hbm,
          i0, i1, g0, g1, o0, o1, si0, si1, sg0, sg1, so0, so1):
        wid = lax.axis_index("s") * NC + lax.axis_index("c")
        wbase = wid * (NSUP * idxc)
        rbase = wid * ROWS_PER_W
        ibufs, gbufs, obufs = (i0, i1), (g0, g1), (o0, o1)
        sis, sgs, sos = (si0, si1), (sg0, sg1), (so0, so1)

        def idx_start(s, buf, sem):
            pltpu.async_copy(idx_hbm.at[pl.ds(wbase + s * idxc, idxc)],
                             buf, sem)

        def idx_wait(s, buf, sem):
            pltpu.make_async_copy(
                idx_hbm.at[pl.ds(wbase + s * idxc, idxc)], buf, sem).wait()

        def gathers_start(ibuf, gbuf, sem):
            for t in range(nstr):
                pltpu.async_copy(
                    tab_hbm.at[ibuf.at[pl.ds(t * slen, slen)]],
                    gbuf.at[pl.ds(t * slen, slen)], sem)

        def gathers_wait(ibuf, gbuf, sem):
            for t in range(nstr):
                pltpu.make_async_copy(
                    tab_hbm.at[ibuf.at[pl.ds(t * slen, slen)]],
                    gbuf.at[pl.ds(t * slen, slen)], sem).wait()

        def out_start(s, obuf, sem):
            pltpu.async_copy(obuf,
                             out_hbm.at[pl.ds(rbase + s * SUPER, SUPER)], sem)

        def out_wait(s, obuf, sem):
            pltpu.make_async_copy(
                obuf, out_hbm.at[pl.ds(rbase + s * SUPER, SUPER)], sem).wait()

        def reduce_step(gbuf, obuf):
            def red_row(r, c):
                qb = r * nfg
                accs = [jnp.zeros((16,), jnp.float32)
                        for _ in range(D // 16)]
                for f in range(nfg):
                    q = qb + f
                    off = 0 if f < npair else D
                    for j in range(D // 16):
                        accs[j] = accs[j] + gbuf[q, pl.ds(off + j * 16, 16)]
                for j in range(D // 16):
                    obuf[r, pl.ds(j * 16, 16)] = accs[j]
                return c
            lax.fori_loop(0, SUPER, red_row, 0)

        pltpu.sync_copy(idx_hbm.at[pl.ds(wbase, idxc)], i0)
        gathers_start(i0, g0, sg0)
        idx_start(1, i1, si1)

        def body2(h, c):
            for par in range(2):
                s = h * 2 + par
                ib, gb, ob = ibufs[par], gbufs[par], obufs[par]
                gathers_wait(ib, gb, sgs[par])

                @pl.when(s + 1 < NSUP)
                def _():
                    idx_wait(s + 1, ibufs[1 - par], sis[1 - par])
                    gathers_start(ibufs[1 - par], gbufs[1 - par],
                                  sgs[1 - par])

                @pl.when(s + 2 < NSUP)
                def _():
                    idx_start(s + 2, ib, sis[par])

                @pl.when(s >= 2)
                def _():
                    out_wait(s - 2, ob, sos[par])

                reduce_step(gb, ob)
                out_start(s, ob, sos[par])
            return c

        lax.fori_loop(0, NSUP // 2, body2, 0)
        out_wait(NSUP - 2, o0, so0)
        out_wait(NSUP - 1, o1, so1)

    return k(flat_idx, tabp)


def _tc_proj(sa, sb, W, b2):
    TILE = 2048

    def body(sa_ref, sb_ref, w_ref, b_ref, o_ref):
        s = sa_ref[:, :D] + sb_ref[:, :D]
        x = jnp.dot(s, w_ref[...],
                    preferred_element_type=jnp.float32) + b_ref[...]
        o_ref[...] = 0.5 * x * (1.0 + lax.erf(x * (1.0 / math.sqrt(2.0))))

    return pl.pallas_call(
        body,
        grid=(B // TILE,),
        in_specs=[
            pl.BlockSpec((TILE, DP), lambda i: (i, 0)),
            pl.BlockSpec((TILE, DP), lambda i: (i, 0)),
            pl.BlockSpec((D, H), lambda i: (0, 0)),
            pl.BlockSpec((1, H), lambda i: (0, 0)),
        ],
        out_specs=pl.BlockSpec((TILE, H), lambda i: (i, 0)),
        out_shape=jax.ShapeDtypeStruct((B, H), jnp.float32),
    )(sa, sb, W, b2)


def _group_idx(e, lo, ni):
    ew = jnp.concatenate([e[:, lo:lo + ni], e[:, lo + HF:lo + HF + ni]],
                         axis=1)
    foffs = jnp.concatenate([jnp.arange(ni, dtype=jnp.int32)] * 2) * VP
    return (ew + foffs[None, :]).reshape(B * 2 * ni)


def kernel(e, tables, W, b):
    tables_t = jnp.transpose(tables, (0, 2, 1))
    idx_a = _group_idx(e, 0, GA)
    idx_b = _group_idx(e, GA, GB)
    tab_a = _pack_table(tables_t, 0, GA)
    sa = _sc_gather_sum(idx_a, tab_a, GA)
    tab_b = _pack_table(tables_t, GA, GB)
    sb = _sc_gather_sum(idx_b, tab_b, GB)
    return _tc_proj(sa, sb, W, b.reshape(1, H))
